# baseline (speedup 1.0000x reference)
import functools

import jax
import jax.numpy as jnp
from jax import lax
from jax.experimental import pallas as pl
from jax.experimental.pallas import tpu as pltpu
from jax.experimental.pallas import tpu_sc as plsc

RES = 200
NPTS = 262144
FEAT = 4
HID = 32
L = 16
NCORES = 2
NSUB = 16
NWORK = NCORES * NSUB
PW = NPTS // NWORK
C = 512
G = C // L
NCHUNK = PW // C
IDXN = 8 * C
R1S = IDXN
R2S = IDXN + G * L
IDXT = R2S + 128


def _sc_body(xyz_hbm, grid_hbm, feat_hbm, xyz_v, idx_v, w_v, rows_v, feat_v, sem):
    wid = lax.axis_index("s") * NCORES + lax.axis_index("c")
    wbase = wid * PW
    lane = lax.iota(jnp.int32, L)
    lane3 = lane * 3
    lane7 = lane * 7
    zero16 = jnp.zeros((L,), jnp.int32)
    idx_v[pl.ds(R2S + 64, 16)] = zero16
    idx_v[pl.ds(R2S + 80, 16)] = zero16
    idx_v[pl.ds(R2S + 96, 16)] = zero16
    idx_v[pl.ds(R2S + 112, 16)] = zero16

    def chunk_body(chunk, _):
        base = wbase + chunk * C
        pltpu.sync_copy(xyz_hbm.at[pl.ds(base * 3, C * 3)], xyz_v)

        def phase_a(g, _):
            off = g * L
            x = plsc.load_gather(xyz_v, [lane3 + off * 3 + 0])
            y = plsc.load_gather(xyz_v, [lane3 + off * 3 + 1])
            z = plsc.load_gather(xyz_v, [lane3 + off * 3 + 2])
            ux = (x + 2.0) * 0.25
            uy = (y + 2.0) * 0.25
            uz = (z + 2.0) * 0.25
            px = ux * 199.0
            py = uy * 199.0
            pz = uz * 199.0
            cx = jnp.minimum(jnp.maximum(px, 0.0), 198.0)
            cy = jnp.minimum(jnp.maximum(py, 0.0), 198.0)
            cz = jnp.minimum(jnp.maximum(pz, 0.0), 198.0)
            ix = cx.astype(jnp.int32)
            iy = cy.astype(jnp.int32)
            iz = cz.astype(jnp.int32)
            fx = px - ix.astype(jnp.float32)
            fy = py - iy.astype(jnp.float32)
            fz = pz - iz.astype(jnp.float32)
            base_idx = (ix * RES + iy) * RES + iz
            gx0 = 1.0 - fx
            gy0 = 1.0 - fy
            gz0 = 1.0 - fz
            w00 = gx0 * gy0
            w01 = gx0 * fy
            w10 = fx * gy0
            w11 = fx * fy
            goff = g * (8 * L)
            corners = (
                (0, w00 * gz0), (1, w00 * fz),
                (RES, w01 * gz0), (RES + 1, w01 * fz),
                (RES * RES, w10 * gz0), (RES * RES + 1, w10 * fz),
                (RES * RES + RES, w11 * gz0), (RES * RES + RES + 1, w11 * fz),
            )
            for c, (doff, wc) in enumerate(corners):
                idx_v[pl.ds(goff + c * L, L)] = base_idx + doff
                w_v[pl.ds(goff + c * L, L)] = wc
            c7 = base_idx + (RES * RES + RES + 1)
            idx_v[pl.ds(R1S + g * L, L)] = c7
            idx_v[pl.ds(R2S + (g // 8) * L, L)] = c7
            plsc.store_scatter(feat_v, [lane7 + off * 7 + 0], ux)
            plsc.store_scatter(feat_v, [lane7 + off * 7 + 1], uy)
            plsc.store_scatter(feat_v, [lane7 + off * 7 + 2], uz)
            return 0

        lax.fori_loop(0, G, phase_a, 0)

        def gather_batch(b, _):
            for j8 in range(8):
                j = b * 8 + j8
                pltpu.async_copy(
                    grid_hbm.at[idx_v.at[pl.ds(j * 256, 128)]],
                    rows_v.at[pl.ds(j * 128, 128), :],
                    sem,
                )
            for j8 in range(8):
                j = b * 8 + j8
                pltpu.make_async_copy(
                    grid_hbm.at[idx_v.at[pl.ds(j * 256, 128)]],
                    rows_v.at[pl.ds(j * 128, 128), :],
                    sem,
                ).wait()
            return 0

        lax.fori_loop(0, IDXN // 128 // 8, gather_batch, 0)
        for ro in (R1S, R1S + 128, R1S + 256, R1S + 384, R2S):
            pltpu.async_copy(
                grid_hbm.at[idx_v.at[pl.ds(2 * ro, 128)]],
                rows_v.at[pl.ds(ro, 128), :], sem)
        for ro in (R1S, R1S + 128, R1S + 256, R1S + 384, R2S):
            pltpu.make_async_copy(
                grid_hbm.at[idx_v.at[pl.ds(2 * ro, 128)]],
                rows_v.at[pl.ds(ro, 128), :], sem).wait()

        def phase_b(g, _):
            off = g * L
            goff = g * (8 * L)
            ws = [w_v[pl.ds(goff + c * L, L)] for c in range(8)]
            r1 = R1S + g * L + lane
            r2 = R2S + (g // 8) * L + lane
            row7 = jnp.where(lane < 14, goff + 7 * L + lane,
                             jnp.where((g % 8) == 7, r2, r1))
            for k in range(FEAT):
                kk = jnp.full((L,), k, jnp.int32)
                acc = ws[0] * plsc.load_gather(rows_v, [goff + lane, kk])
                for c in range(1, 7):
                    acc = acc + ws[c] * plsc.load_gather(
                        rows_v, [goff + c * L + lane, kk])
                acc = acc + ws[7] * plsc.load_gather(rows_v, [row7, kk])
                plsc.store_scatter(feat_v, [lane7 + off * 7 + (3 + k)], acc)
            return 0

        lax.fori_loop(0, G, phase_b, 0)
        pltpu.sync_copy(feat_v, feat_hbm.at[pl.ds(base * 7, C * 7)])
        return 0

    lax.fori_loop(0, NCHUNK, chunk_body, 0)


_sc_featurize = functools.partial(
    pl.kernel,
    out_type=jax.ShapeDtypeStruct((NPTS * 7,), jnp.float32),
    mesh=plsc.VectorSubcoreMesh(
        core_axis_name="c", subcore_axis_name="s",
        num_cores=NCORES, num_subcores=NSUB),
    scratch_types=[
        pltpu.VMEM((C * 3,), jnp.float32),
        pltpu.VMEM((2 * R2S + 128,), jnp.int32),
        pltpu.VMEM((IDXN,), jnp.float32),
        pltpu.VMEM((IDXT, FEAT), jnp.float32),
        pltpu.VMEM((C * 7,), jnp.float32),
        pltpu.SemaphoreType.DMA,
    ],
    compiler_params=pltpu.CompilerParams(
        needs_layout_passes=False, use_tc_tiling_on_sc=False),
)(_sc_body)


def _mlp_body(feat_ref, w0_ref, w1s_ref, w1c_ref, rgb_ref, sdf_ref):
    ft = feat_ref[...]
    h = jnp.maximum(
        jnp.dot(ft, w0_ref[...], preferred_element_type=jnp.float32), 0.0)
    sdf_ref[...] = jnp.dot(h[:, :HID], w1s_ref[...],
                           preferred_element_type=jnp.float32)
    rgb_ref[...] = jnp.dot(h[:, HID:], w1c_ref[...],
                           preferred_element_type=jnp.float32)


def _tc_mlp(feat, w0, w1s, w1c):
    B = 4096
    return pl.pallas_call(
        _mlp_body,
        grid=(NPTS // B,),
        in_specs=[
            pl.BlockSpec((B, 7), lambda i: (i, 0)),
            pl.BlockSpec((7, 2 * HID), lambda i: (0, 0)),
            pl.BlockSpec((HID, 1), lambda i: (0, 0)),
            pl.BlockSpec((HID, 3), lambda i: (0, 0)),
        ],
        out_specs=[
            pl.BlockSpec((B, 3), lambda i: (i, 0)),
            pl.BlockSpec((B, 1), lambda i: (i, 0)),
        ],
        out_shape=[
            jax.ShapeDtypeStruct((NPTS, 3), jnp.float32),
            jax.ShapeDtypeStruct((NPTS, 1), jnp.float32),
        ],
    )(feat, w0, w1s, w1c)


def kernel(xyz, grid, sdf_w0, sdf_w1, col_w0, col_w1):
    feat = _sc_featurize(xyz.reshape(-1), grid).reshape(NPTS, 7)
    w0 = jnp.concatenate(
        [jnp.concatenate([sdf_w0[FEAT:], sdf_w0[:FEAT]], axis=0), col_w0],
        axis=1)
    rgb, sdf = _tc_mlp(feat, w0, sdf_w1, col_w1)
    return rgb, sdf.reshape(-1)
